# Initial kernel scaffold; baseline (speedup 1.0000x reference)
import jax
import jax.numpy as jnp
from jax.experimental import pallas as pl


def kernel(atom_type, atom_pos, edge_src, edge_dst, edge_cell_shift, cell, image_index, W_in, W_r1, W_r2, W_sh, W_sc, W_node, W_v1, W_v2, W_f1, W_f2):
    raise NotImplementedError("write your pallas kernel here")


# speedup vs baseline: 1.4600x; 1.4600x over previous
import functools

import jax
import jax.numpy as jnp
from jax.experimental import pallas as pl
from jax.experimental.pallas import tpu as pltpu

N = 50000
E = 800000
B = 64
ELEM = 2
D = 32
EMB = 16
NB = 8
LAYERS = 3
RMAX = 4.0
P = 6
AVG_NEI = 20.0
NEUR = 64
SH_DIM = 4

NBLK_N = 2000
NBLK_E = 4000


def _silu(z):
    return z * jax.nn.sigmoid(z)


def _k1_body(at_ref, img_ref, pos_ref, cell9_ref, win_ref, a_ref, x0_ref):
    at = at_ref[0, 0, :]
    img = img_ref[0, 0, :]
    pos = pos_ref[...]
    blk = at.shape[0]
    oh = (jax.lax.broadcasted_iota(jnp.int32, (blk, B), 1) == img[:, None]).astype(jnp.float32)
    cellrow = jnp.dot(oh, cell9_ref[...], preferred_element_type=jnp.float32, precision=jax.lax.Precision.HIGHEST)
    a_ref[...] = jnp.concatenate(
        [pos, cellrow, jnp.zeros((blk, 4), jnp.float32)], axis=1)
    oh2 = (jax.lax.broadcasted_iota(jnp.int32, (blk, ELEM), 1) == at[:, None]).astype(jnp.float32)
    x0_ref[...] = jnp.dot(oh2, win_ref[...], preferred_element_type=jnp.float32)


def _build_tables(atom_type, image_index, atom_pos, cell9, W_in):
    nblk = N // NBLK_N
    at3 = atom_type.reshape(nblk, 1, NBLK_N).astype(jnp.int32)
    img3 = image_index.reshape(nblk, 1, NBLK_N).astype(jnp.int32)
    return pl.pallas_call(
        _k1_body,
        grid=(nblk,),
        in_specs=[
            pl.BlockSpec((1, 1, NBLK_N), lambda i: (i, 0, 0)),
            pl.BlockSpec((1, 1, NBLK_N), lambda i: (i, 0, 0)),
            pl.BlockSpec((NBLK_N, 3), lambda i: (i, 0)),
            pl.BlockSpec((B, 9), lambda i: (0, 0)),
            pl.BlockSpec((ELEM, D), lambda i: (0, 0)),
        ],
        out_specs=[
            pl.BlockSpec((NBLK_N, 16), lambda i: (i, 0)),
            pl.BlockSpec((NBLK_N, D), lambda i: (i, 0)),
        ],
        out_shape=[
            jax.ShapeDtypeStruct((N, 16), jnp.float32),
            jax.ShapeDtypeStruct((N, D), jnp.float32),
        ],
    )(at3, img3, atom_pos, cell9, W_in)


def _k3_body(srcg_ref, dstg_ref, ecs_ref, wr1_ref, wr2_ref, wsh_ref, w_ref):
    srcg = srcg_ref[...]
    dstg = dstg_ref[...]
    ecs = ecs_ref[...]
    be = srcg.shape[0]
    vs = []
    for j in range(3):
        shift = (ecs[:, 0:1] * srcg[:, 3 + j:4 + j]
                 + ecs[:, 1:2] * srcg[:, 6 + j:7 + j]
                 + ecs[:, 2:3] * srcg[:, 9 + j:10 + j])
        vs.append(dstg[:, j:j + 1] - srcg[:, j:j + 1] + shift)
    r2 = vs[0] * vs[0] + vs[1] * vs[1] + vs[2] * vs[2]
    r = jnp.sqrt(r2)
    rinv = 1.0 / (r + 1e-12)
    sh = jnp.concatenate(
        [jnp.ones((be, 1), jnp.float32),
         jnp.sqrt(3.0) * vs[0] * rinv,
         jnp.sqrt(3.0) * vs[1] * rinv,
         jnp.sqrt(3.0) * vs[2] * rinv], axis=1)
    u_full = r * (1.0 / RMAX)
    a = jnp.float32(jnp.pi) * jnp.minimum(u_full, 1.0)
    bb = jnp.minimum(a, jnp.float32(jnp.pi) - a)
    b2 = bb * bb
    sb = bb * (1.0 + b2 * (-1.0 / 6.0 + b2 * (1.0 / 120.0 + b2 * (-1.0 / 5040.0
         + b2 * (1.0 / 362880.0 - b2 * (1.0 / 39916800.0))))))
    cb = 1.0 + b2 * (-0.5 + b2 * (1.0 / 24.0 + b2 * (-1.0 / 720.0
         + b2 * (1.0 / 40320.0 + b2 * (-1.0 / 3628800.0 + b2 * (1.0 / 479001600.0))))))
    s1 = sb
    c1 = jnp.where(a > jnp.float32(jnp.pi / 2), -cb, cb)
    sn, cn = s1, c1
    sins = [s1]
    for _ in range(NB - 1):
        sn, cn = sn * c1 + cn * s1, cn * c1 - sn * s1
        sins.append(sn)
    sin_mat = jnp.concatenate(sins, axis=1)
    bes = jnp.sqrt(2.0 / RMAX) * sin_mat * rinv
    u = r / RMAX
    u2 = u * u
    u3 = u2 * u
    u6 = u3 * u3
    f_cut = (1.0 - ((P + 1.0) * (P + 2.0) / 2.0) * u6
             + P * (P + 2.0) * u6 * u - (P * (P + 1.0) / 2.0) * u6 * u2)
    f_cut = jnp.where(u < 1.0, f_cut, 0.0)
    radial = bes * f_cut
    for l in range(LAYERS):
        h = _silu(jnp.dot(radial, wr1_ref[l], preferred_element_type=jnp.float32))
        rw = jnp.dot(h, wr2_ref[l], preferred_element_type=jnp.float32)
        eg = jnp.dot(sh, wsh_ref[l], preferred_element_type=jnp.float32)
        w_ref[l, :, :] = rw * eg * (1.0 / AVG_NEI)


def _edge_weights(srcg, dstg, ecs4, W_r1, W_r2, W_sh):
    nblk = E // NBLK_E
    return pl.pallas_call(
        _k3_body,
        grid=(nblk,),
        in_specs=[
            pl.BlockSpec((NBLK_E, 16), lambda i: (i, 0)),
            pl.BlockSpec((NBLK_E, 4), lambda i: (i, 0)),
            pl.BlockSpec((NBLK_E, 4), lambda i: (i, 0)),
            pl.BlockSpec((LAYERS, NB, NEUR), lambda i: (0, 0, 0)),
            pl.BlockSpec((LAYERS, NEUR, D), lambda i: (0, 0, 0)),
            pl.BlockSpec((LAYERS, SH_DIM, D), lambda i: (0, 0, 0)),
        ],
        out_specs=pl.BlockSpec((LAYERS, NBLK_E, D), lambda i: (0, i, 0)),
        out_shape=jax.ShapeDtypeStruct((LAYERS, E, D), jnp.float32),
    )(srcg, dstg, ecs4, W_r1, W_r2, W_sh)


def _k5_body(x_ref, agg_ref, wsc_ref, wnode_ref, out_ref):
    x = x_ref[...]
    agg = agg_ref[...]
    sc = jnp.dot(x, wsc_ref[...], preferred_element_type=jnp.float32)
    new = _silu(jnp.dot(agg + sc, wnode_ref[...], preferred_element_type=jnp.float32))
    out_ref[...] = x + new


def _node_update(x, agg, wsc, wnode):
    nblk = N // NBLK_N
    return pl.pallas_call(
        _k5_body,
        grid=(nblk,),
        in_specs=[
            pl.BlockSpec((NBLK_N, D), lambda i: (i, 0)),
            pl.BlockSpec((NBLK_N, D), lambda i: (i, 0)),
            pl.BlockSpec((D, D), lambda i: (0, 0)),
            pl.BlockSpec((D, D), lambda i: (0, 0)),
        ],
        out_specs=pl.BlockSpec((NBLK_N, D), lambda i: (i, 0)),
        out_shape=jax.ShapeDtypeStruct((N, D), jnp.float32),
    )(x, agg, wsc, wnode)


def _k6_body(img_ref, x_ref, wv1_ref, wv2_ref, wf1_ref, wf2_ref,
             fermi_ref, vac_ref, s_ref, s2_ref, c_ref):
    i = pl.program_id(0)
    nsteps = pl.num_programs(0)

    @pl.when(i == 0)
    def _():
        s_ref[...] = jnp.zeros_like(s_ref)
        s2_ref[...] = jnp.zeros_like(s2_ref)
        c_ref[...] = jnp.zeros_like(c_ref)

    img = img_ref[0, 0, :]
    x = x_ref[...]
    blk = x.shape[0]
    oh = (jax.lax.broadcasted_iota(jnp.int32, (B, blk), 0) == img[None, :]).astype(jnp.float32)
    s_ref[...] += jnp.dot(oh, x, preferred_element_type=jnp.float32, precision=jax.lax.Precision.HIGHEST)
    s2_ref[...] += jnp.dot(oh, x * x, preferred_element_type=jnp.float32, precision=jax.lax.Precision.HIGHEST)
    c_ref[...] += jnp.sum(oh, axis=1, keepdims=True)

    @pl.when(i == nsteps - 1)
    def _():
        cnt = jnp.maximum(c_ref[...], 1.0)
        mean = s_ref[...] / cnt
        mean2 = s2_ref[...] / cnt
        std = jnp.sqrt(jnp.maximum(mean2 - mean * mean, 0.0) + 1e-12)
        fermi_ref[...] = jnp.dot(
            jnp.dot(std, wf1_ref[...], preferred_element_type=jnp.float32),
            wf2_ref[...], preferred_element_type=jnp.float32)
        vac_ref[...] = jnp.dot(
            jnp.dot(mean, wv1_ref[...], preferred_element_type=jnp.float32),
            wv2_ref[...], preferred_element_type=jnp.float32)


def _readout(image_index, x, W_v1, W_v2, W_f1, W_f2):
    nblk = N // NBLK_N
    img3 = image_index.reshape(nblk, 1, NBLK_N).astype(jnp.int32)
    return pl.pallas_call(
        _k6_body,
        grid=(nblk,),
        in_specs=[
            pl.BlockSpec((1, 1, NBLK_N), lambda i: (i, 0, 0)),
            pl.BlockSpec((NBLK_N, D), lambda i: (i, 0)),
            pl.BlockSpec((D, EMB), lambda i: (0, 0)),
            pl.BlockSpec((EMB, 1), lambda i: (0, 0)),
            pl.BlockSpec((D, EMB), lambda i: (0, 0)),
            pl.BlockSpec((EMB, 1), lambda i: (0, 0)),
        ],
        out_specs=[
            pl.BlockSpec((B, 1), lambda i: (0, 0)),
            pl.BlockSpec((B, 1), lambda i: (0, 0)),
        ],
        out_shape=[
            jax.ShapeDtypeStruct((B, 1), jnp.float32),
            jax.ShapeDtypeStruct((B, 1), jnp.float32),
        ],
        scratch_shapes=[
            pltpu.VMEM((B, D), jnp.float32),
            pltpu.VMEM((B, D), jnp.float32),
            pltpu.VMEM((B, 1), jnp.float32),
        ],
    )(img3, x, W_v1, W_v2, W_f1, W_f2)


def kernel(atom_type, atom_pos, edge_src, edge_dst, edge_cell_shift, cell,
           image_index, W_in, W_r1, W_r2, W_sh, W_sc, W_node, W_v1, W_v2,
           W_f1, W_f2):
    atom_type = atom_type.astype(jnp.int32)
    edge_src = edge_src.astype(jnp.int32)
    edge_dst = edge_dst.astype(jnp.int32)
    image_index = image_index.astype(jnp.int32)
    cell9 = cell.reshape(B, 9).astype(jnp.float32)
    ecs4 = jnp.concatenate(
        [edge_cell_shift, jnp.zeros((E, 1), jnp.float32)], axis=1)

    a_tab, x = _build_tables(atom_type, image_index, atom_pos, cell9, W_in)

    srcg = a_tab[edge_src]
    dstg = a_tab[edge_dst, :4]

    if False:
        vec = dstg[:, :3] - srcg[:, :3] + jnp.einsum('ni,nij->nj', edge_cell_shift, srcg[:, 3:12].reshape(E, 3, 3))
        r = jnp.linalg.norm(vec, axis=1)
        unit = vec / (r[:, None] + 1e-12)
        sh = jnp.concatenate([jnp.ones((E, 1), jnp.float32), jnp.sqrt(3.0) * unit], axis=1)
        nn = jnp.arange(1, NB + 1, dtype=jnp.float32)
        bes = jnp.sqrt(2.0 / RMAX) * jnp.sin(nn[None, :] * jnp.pi * r[:, None] / RMAX) / (r[:, None] + 1e-12)
        u = r / RMAX
        f_cut = (1.0 - ((P + 1.0) * (P + 2.0) / 2.0) * u ** P
                 + P * (P + 2.0) * u ** (P + 1) - (P * (P + 1.0) / 2.0) * u ** (P + 2))
        f_cut = jnp.where(u < 1.0, f_cut, 0.0)
        radial = bes * f_cut[:, None]
        w_all = []
        for l in range(LAYERS):
            rw = jax.nn.silu(radial @ W_r1[l]) @ W_r2[l]
            eg = sh @ W_sh[l]
            w_all.append(rw * eg * (1.0 / AVG_NEI))
        w_all = jnp.stack(w_all)
    else:
        w_all = _edge_weights(srcg, dstg, ecs4, W_r1, W_r2, W_sh)

    for l in range(LAYERS):
        msg = x[edge_src] * w_all[l]
        agg = jnp.zeros_like(x).at[edge_dst].add(msg)
        if False:
            x = x + jax.nn.silu((agg + x @ W_sc[l]) @ W_node[l])
        else:
            x = _node_update(x, agg, W_sc[l], W_node[l])

    if False:
        cnt = jnp.zeros((B,), dtype=jnp.float32).at[image_index].add(1.0)
        cnt = jnp.maximum(cnt, 1.0)
        mean = jnp.zeros((B, D), dtype=jnp.float32).at[image_index].add(x) / cnt[:, None]
        mean2 = jnp.zeros((B, D), dtype=jnp.float32).at[image_index].add(x * x) / cnt[:, None]
        std = jnp.sqrt(jnp.maximum(mean2 - mean * mean, 0.0) + 1e-12)
        vac = (mean @ W_v1) @ W_v2
        fermi = (std @ W_f1) @ W_f2
    else:
        fermi, vac = _readout(image_index, x, W_v1, W_v2, W_f1, W_f2)
    return fermi, vac


# speedup vs baseline: 2.1134x; 1.4475x over previous
import functools

import jax
from jax import lax
import jax.numpy as jnp
from jax.experimental import pallas as pl
from jax.experimental.pallas import tpu as pltpu
from jax.experimental.pallas import tpu_sc as plsc

N = 50000
E = 800000
B = 64
ELEM = 2
D = 32
EMB = 16
NB = 8
LAYERS = 3
RMAX = 4.0
P = 6
AVG_NEI = 20.0
NEUR = 64
SH_DIM = 4

NBLK_N = 2000
NBLK_E = 4096

NC = 2
NS = 16
NW = NC * NS
CH = 512
RW = CH // 128
CPW = 50
E_PAD = NW * CH * CPW
ROWS_PW = CPW * RW
ZR = N // NS


def _silu(z):
    return z * jax.nn.sigmoid(z)


def _k1_body(at_ref, img_ref, pos_ref, cell9_ref, win_ref, a_ref, x0_ref):
    at = at_ref[0, 0, :]
    img = img_ref[0, 0, :]
    pos = pos_ref[...]
    blk = at.shape[0]
    oh = (jax.lax.broadcasted_iota(jnp.int32, (blk, B), 1) == img[:, None]).astype(jnp.float32)
    cellrow = jnp.dot(oh, cell9_ref[...], preferred_element_type=jnp.float32, precision=jax.lax.Precision.HIGHEST)
    a_ref[...] = jnp.concatenate(
        [pos, cellrow, jnp.zeros((blk, 4), jnp.float32)], axis=1)
    oh2 = (jax.lax.broadcasted_iota(jnp.int32, (blk, ELEM), 1) == at[:, None]).astype(jnp.float32)
    x0_ref[...] = jnp.dot(oh2, win_ref[...], preferred_element_type=jnp.float32)


def _build_tables(atom_type, image_index, atom_pos, cell9, W_in):
    nblk = N // NBLK_N
    at3 = atom_type.reshape(nblk, 1, NBLK_N).astype(jnp.int32)
    img3 = image_index.reshape(nblk, 1, NBLK_N).astype(jnp.int32)
    return pl.pallas_call(
        _k1_body,
        grid=(nblk,),
        in_specs=[
            pl.BlockSpec((1, 1, NBLK_N), lambda i: (i, 0, 0)),
            pl.BlockSpec((1, 1, NBLK_N), lambda i: (i, 0, 0)),
            pl.BlockSpec((NBLK_N, 3), lambda i: (i, 0)),
            pl.BlockSpec((B, 9), lambda i: (0, 0)),
            pl.BlockSpec((ELEM, D), lambda i: (0, 0)),
        ],
        out_specs=[
            pl.BlockSpec((NBLK_N, 16), lambda i: (i, 0)),
            pl.BlockSpec((NBLK_N, D), lambda i: (i, 0)),
        ],
        out_shape=[
            jax.ShapeDtypeStruct((N, 16), jnp.float32),
            jax.ShapeDtypeStruct((N, D), jnp.float32),
        ],
    )(at3, img3, atom_pos, cell9, W_in)


def _k3_body(srcg_ref, dstg_ref, ecs_ref, wr1_ref, wr2_ref, wsh_ref, w_ref):
    srcg = srcg_ref[...]
    dstg = dstg_ref[...]
    ecs = ecs_ref[...]
    be = srcg.shape[0]
    vs = []
    for j in range(3):
        shift = (ecs[:, 0:1] * srcg[:, 3 + j:4 + j]
                 + ecs[:, 1:2] * srcg[:, 6 + j:7 + j]
                 + ecs[:, 2:3] * srcg[:, 9 + j:10 + j])
        vs.append(dstg[:, j:j + 1] - srcg[:, j:j + 1] + shift)
    r2 = vs[0] * vs[0] + vs[1] * vs[1] + vs[2] * vs[2]
    r = jnp.sqrt(r2)
    rinv = 1.0 / (r + 1e-12)
    sh = jnp.concatenate(
        [jnp.ones((be, 1), jnp.float32),
         jnp.sqrt(3.0) * vs[0] * rinv,
         jnp.sqrt(3.0) * vs[1] * rinv,
         jnp.sqrt(3.0) * vs[2] * rinv], axis=1)
    u_full = r * (1.0 / RMAX)
    a = jnp.float32(jnp.pi) * jnp.minimum(u_full, 1.0)
    bb = jnp.minimum(a, jnp.float32(jnp.pi) - a)
    b2 = bb * bb
    sb = bb * (1.0 + b2 * (-1.0 / 6.0 + b2 * (1.0 / 120.0 + b2 * (-1.0 / 5040.0
         + b2 * (1.0 / 362880.0 - b2 * (1.0 / 39916800.0))))))
    cb = 1.0 + b2 * (-0.5 + b2 * (1.0 / 24.0 + b2 * (-1.0 / 720.0
         + b2 * (1.0 / 40320.0 + b2 * (-1.0 / 3628800.0 + b2 * (1.0 / 479001600.0))))))
    s1 = sb
    c1 = jnp.where(a > jnp.float32(jnp.pi / 2), -cb, cb)
    sn, cn = s1, c1
    sins = [s1]
    for _ in range(NB - 1):
        sn, cn = sn * c1 + cn * s1, cn * c1 - sn * s1
        sins.append(sn)
    sin_mat = jnp.concatenate(sins, axis=1)
    bes = jnp.sqrt(2.0 / RMAX) * sin_mat * rinv
    u = r / RMAX
    u2 = u * u
    u3 = u2 * u
    u6 = u3 * u3
    f_cut = (1.0 - ((P + 1.0) * (P + 2.0) / 2.0) * u6
             + P * (P + 2.0) * u6 * u - (P * (P + 1.0) / 2.0) * u6 * u2)
    f_cut = jnp.where(u < 1.0, f_cut, 0.0)
    radial = bes * f_cut
    row0 = pl.program_id(0) * be
    valid = ((jax.lax.broadcasted_iota(jnp.int32, (be, 1), 0) + row0) < E
             ).astype(jnp.float32)
    for l in range(LAYERS):
        h = _silu(jnp.dot(radial, wr1_ref[l], preferred_element_type=jnp.float32))
        rw = jnp.dot(h, wr2_ref[l], preferred_element_type=jnp.float32)
        eg = jnp.dot(sh, wsh_ref[l], preferred_element_type=jnp.float32)
        w_ref[l, :, :] = rw * eg * ((1.0 / AVG_NEI) * valid)


def _edge_weights(srcg, dstg, ecs4, W_r1, W_r2, W_sh):
    nblk = E_PAD // NBLK_E
    return pl.pallas_call(
        _k3_body,
        grid=(nblk,),
        in_specs=[
            pl.BlockSpec((NBLK_E, 16), lambda i: (i, 0)),
            pl.BlockSpec((NBLK_E, 16), lambda i: (i, 0)),
            pl.BlockSpec((NBLK_E, 4), lambda i: (i, 0)),
            pl.BlockSpec((LAYERS, NB, NEUR), lambda i: (0, 0, 0)),
            pl.BlockSpec((LAYERS, NEUR, D), lambda i: (0, 0, 0)),
            pl.BlockSpec((LAYERS, SH_DIM, D), lambda i: (0, 0, 0)),
        ],
        out_specs=pl.BlockSpec((LAYERS, NBLK_E, D), lambda i: (0, i, 0)),
        out_shape=jax.ShapeDtypeStruct((LAYERS, E_PAD, D), jnp.float32),
    )(srcg, dstg, ecs4, W_r1, W_r2, W_sh)


def _sc_gather_geom(a_tab, src2d, dst2d):
    mesh = plsc.VectorSubcoreMesh(core_axis_name="c", subcore_axis_name="s")

    @functools.partial(
        pl.kernel,
        out_type=[jax.ShapeDtypeStruct((E_PAD, 16), jnp.float32),
                  jax.ShapeDtypeStruct((E_PAD, 16), jnp.float32)],
        mesh=mesh,
        compiler_params=pltpu.CompilerParams(use_tc_tiling_on_sc=False),
        scratch_types=[
            pltpu.VMEM((RW, 128), jnp.int32),
            pltpu.VMEM((RW, 128), jnp.int32),
            pltpu.VMEM((CH, 16), jnp.float32),
            pltpu.VMEM((CH, 16), jnp.float32),
            pltpu.SemaphoreType.DMA,
            pltpu.SemaphoreType.DMA,
        ])
    def k(a_hbm, src_hbm, dst_hbm, srcg_hbm, dstg_hbm,
          idxs, idxd, bs, bd, sem_s, sem_d):
        wid = lax.axis_index("s") * NC + lax.axis_index("c")

        def body(c, _):
            row0 = wid * ROWS_PW + c * RW
            e0 = row0 * 128
            pltpu.sync_copy(src_hbm.at[pl.ds(row0, RW)], idxs)
            pltpu.sync_copy(dst_hbm.at[pl.ds(row0, RW)], idxd)
            cps = [pltpu.async_copy(a_hbm.at[idxs.at[j]],
                                    bs.at[pl.ds(j * 128, 128)], sem_s)
                   for j in range(RW)]
            cpd = [pltpu.async_copy(a_hbm.at[idxd.at[j]],
                                    bd.at[pl.ds(j * 128, 128)], sem_d)
                   for j in range(RW)]
            for cp in cps + cpd:
                cp.wait()
            pltpu.sync_copy(bs, srcg_hbm.at[pl.ds(e0, CH)])
            pltpu.sync_copy(bd, dstg_hbm.at[pl.ds(e0, CH)])
            return 0

        lax.fori_loop(0, CPW, body, 0)

    return k(a_tab, src2d, dst2d)


HALF = N // 2
HALF_PAD = 26000
ZR2 = HALF_PAD // NS
CPW2 = E_PAD // (NS * CH)
ROWS_PS = CPW2 * RW


def _sc_layer(x, w_all, l, src2d, dst2d):
    mesh = plsc.VectorSubcoreMesh(core_axis_name="c", subcore_axis_name="s")

    @functools.partial(
        pl.kernel,
        out_type=jax.ShapeDtypeStruct((2 * HALF_PAD, D), jnp.float32),
        mesh=mesh,
        compiler_params=pltpu.CompilerParams(use_tc_tiling_on_sc=False),
        scratch_types=[
            pltpu.VMEM((RW, 128), jnp.int32),
            pltpu.VMEM((RW, 128), jnp.int32),
            pltpu.VMEM((CH, D), jnp.float32),
            pltpu.VMEM((CH, D), jnp.float32),
            pltpu.VMEM((128, D), jnp.float32),
            pltpu.VMEM_SHARED((HALF_PAD, D), jnp.float32),
            pltpu.SemaphoreType.DMA,
        ])
    def k(x_hbm, w_hbm, src_hbm, dst_hbm, out_hbm,
          idxs, idxd, xr, wr, zb, acc, sem):
        cid = lax.axis_index("c")
        sid = lax.axis_index("s")
        lo = cid * HALF

        def zvec(i, _):
            zb[i, pl.ds(0, 16)] = jnp.zeros((16,), jnp.float32)
            zb[i, pl.ds(16, 16)] = jnp.zeros((16,), jnp.float32)
            return 0
        lax.fori_loop(0, 128, zvec, 0, unroll=8)
        base = sid * ZR2
        nfull = ZR2 // 128
        rem = ZR2 - nfull * 128
        def zcopy(i, _):
            pltpu.sync_copy(zb, acc.at[pl.ds(base + i * 128, 128)])
            return 0
        lax.fori_loop(0, nfull, zcopy, 0)
        if rem:
            pltpu.sync_copy(zb.at[pl.ds(0, rem)],
                            acc.at[pl.ds(base + nfull * 128, rem)])
        plsc.subcore_barrier()

        def body(c, _):
            row0 = sid * ROWS_PS + c * RW
            e0 = row0 * 128
            pltpu.sync_copy(src_hbm.at[pl.ds(row0, RW)], idxs)
            pltpu.sync_copy(dst_hbm.at[pl.ds(row0, RW)], idxd)
            cps = [pltpu.async_copy(x_hbm.at[idxs.at[j]],
                                    xr.at[pl.ds(j * 128, 128)], sem)
                   for j in range(RW)]
            pltpu.sync_copy(w_hbm.at[l, pl.ds(e0, CH)], wr)
            def remap(i, _):
                v = idxd[i, pl.ds(0, 16)] - lo
                ok = (v >= 0) & (v < HALF)
                idxd[i, pl.ds(0, 16)] = jnp.where(ok, v, HALF)
                v = idxd[i, pl.ds(16, 16)] - lo
                ok = (v >= 0) & (v < HALF)
                idxd[i, pl.ds(16, 16)] = jnp.where(ok, v, HALF)
                v = idxd[i, pl.ds(32, 16)] - lo
                ok = (v >= 0) & (v < HALF)
                idxd[i, pl.ds(32, 16)] = jnp.where(ok, v, HALF)
                v = idxd[i, pl.ds(48, 16)] - lo
                ok = (v >= 0) & (v < HALF)
                idxd[i, pl.ds(48, 16)] = jnp.where(ok, v, HALF)
                v = idxd[i, pl.ds(64, 16)] - lo
                ok = (v >= 0) & (v < HALF)
                idxd[i, pl.ds(64, 16)] = jnp.where(ok, v, HALF)
                v = idxd[i, pl.ds(80, 16)] - lo
                ok = (v >= 0) & (v < HALF)
                idxd[i, pl.ds(80, 16)] = jnp.where(ok, v, HALF)
                v = idxd[i, pl.ds(96, 16)] - lo
                ok = (v >= 0) & (v < HALF)
                idxd[i, pl.ds(96, 16)] = jnp.where(ok, v, HALF)
                v = idxd[i, pl.ds(112, 16)] - lo
                ok = (v >= 0) & (v < HALF)
                idxd[i, pl.ds(112, 16)] = jnp.where(ok, v, HALF)
                return 0
            lax.fori_loop(0, RW, remap, 0)
            for cp in cps:
                cp.wait()
            def mul(i, _):
                xr[i, pl.ds(0, 16)] = xr[i, pl.ds(0, 16)] * wr[i, pl.ds(0, 16)]
                xr[i, pl.ds(16, 16)] = xr[i, pl.ds(16, 16)] * wr[i, pl.ds(16, 16)]
                return 0
            lax.fori_loop(0, CH, mul, 0, unroll=8)
            for j in range(RW):
                pltpu.sync_copy(xr.at[pl.ds(j * 128, 128)],
                                acc.at[idxd.at[j]], add=True)
            return 0

        lax.fori_loop(0, CPW2, body, 0)
        plsc.subcore_barrier()
        pltpu.sync_copy(acc.at[pl.ds(base, ZR2)],
                        out_hbm.at[pl.ds(cid * HALF_PAD + base, ZR2)])

    return k(x, w_all, src2d, dst2d)


NBLK_H = 1000


def _k5_body(x_ref, agg_ref, wsc_ref, wnode_ref, out_ref):
    x = x_ref[...]
    agg = agg_ref[...]
    sc = jnp.dot(x, wsc_ref[...], preferred_element_type=jnp.float32)
    new = _silu(jnp.dot(agg + sc, wnode_ref[...], preferred_element_type=jnp.float32))
    out_ref[...] = x + new


def _node_update(x, agg_pad, wsc, wnode):
    nblk = N // NBLK_H
    hb = HALF // NBLK_H
    return pl.pallas_call(
        _k5_body,
        grid=(nblk,),
        in_specs=[
            pl.BlockSpec((NBLK_H, D), lambda i: (i, 0)),
            pl.BlockSpec((NBLK_H, D),
                         lambda i: (jnp.where(i < hb, i, i + 1), 0)),
            pl.BlockSpec((D, D), lambda i: (0, 0)),
            pl.BlockSpec((D, D), lambda i: (0, 0)),
        ],
        out_specs=pl.BlockSpec((NBLK_H, D), lambda i: (i, 0)),
        out_shape=jax.ShapeDtypeStruct((N, D), jnp.float32),
    )(x, agg_pad, wsc, wnode)


def _k6_body(img_ref, x_ref, wv1_ref, wv2_ref, wf1_ref, wf2_ref,
             fermi_ref, vac_ref, s_ref, s2_ref, c_ref):
    i = pl.program_id(0)
    nsteps = pl.num_programs(0)

    @pl.when(i == 0)
    def _():
        s_ref[...] = jnp.zeros_like(s_ref)
        s2_ref[...] = jnp.zeros_like(s2_ref)
        c_ref[...] = jnp.zeros_like(c_ref)

    img = img_ref[0, 0, :]
    x = x_ref[...]
    blk = x.shape[0]
    oh = (jax.lax.broadcasted_iota(jnp.int32, (B, blk), 0) == img[None, :]).astype(jnp.float32)
    s_ref[...] += jnp.dot(oh, x, preferred_element_type=jnp.float32, precision=jax.lax.Precision.HIGHEST)
    s2_ref[...] += jnp.dot(oh, x * x, preferred_element_type=jnp.float32, precision=jax.lax.Precision.HIGHEST)
    c_ref[...] += jnp.sum(oh, axis=1, keepdims=True)

    @pl.when(i == nsteps - 1)
    def _():
        cnt = jnp.maximum(c_ref[...], 1.0)
        mean = s_ref[...] / cnt
        mean2 = s2_ref[...] / cnt
        std = jnp.sqrt(jnp.maximum(mean2 - mean * mean, 0.0) + 1e-12)
        fermi_ref[...] = jnp.dot(
            jnp.dot(std, wf1_ref[...], preferred_element_type=jnp.float32),
            wf2_ref[...], preferred_element_type=jnp.float32)
        vac_ref[...] = jnp.dot(
            jnp.dot(mean, wv1_ref[...], preferred_element_type=jnp.float32),
            wv2_ref[...], preferred_element_type=jnp.float32)


def _readout(image_index, x, W_v1, W_v2, W_f1, W_f2):
    nblk = N // NBLK_N
    img3 = image_index.reshape(nblk, 1, NBLK_N).astype(jnp.int32)
    return pl.pallas_call(
        _k6_body,
        grid=(nblk,),
        in_specs=[
            pl.BlockSpec((1, 1, NBLK_N), lambda i: (i, 0, 0)),
            pl.BlockSpec((NBLK_N, D), lambda i: (i, 0)),
            pl.BlockSpec((D, EMB), lambda i: (0, 0)),
            pl.BlockSpec((EMB, 1), lambda i: (0, 0)),
            pl.BlockSpec((D, EMB), lambda i: (0, 0)),
            pl.BlockSpec((EMB, 1), lambda i: (0, 0)),
        ],
        out_specs=[
            pl.BlockSpec((B, 1), lambda i: (0, 0)),
            pl.BlockSpec((B, 1), lambda i: (0, 0)),
        ],
        out_shape=[
            jax.ShapeDtypeStruct((B, 1), jnp.float32),
            jax.ShapeDtypeStruct((B, 1), jnp.float32),
        ],
        scratch_shapes=[
            pltpu.VMEM((B, D), jnp.float32),
            pltpu.VMEM((B, D), jnp.float32),
            pltpu.VMEM((B, 1), jnp.float32),
        ],
    )(img3, x, W_v1, W_v2, W_f1, W_f2)


def kernel(atom_type, atom_pos, edge_src, edge_dst, edge_cell_shift, cell,
           image_index, W_in, W_r1, W_r2, W_sh, W_sc, W_node, W_v1, W_v2,
           W_f1, W_f2):
    atom_type = atom_type.astype(jnp.int32)
    image_index = image_index.astype(jnp.int32)
    cell9 = cell.reshape(B, 9).astype(jnp.float32)
    pad = E_PAD - E
    src2d = jnp.concatenate(
        [edge_src.astype(jnp.int32), jnp.zeros((pad,), jnp.int32)]
    ).reshape(E_PAD // 128, 128)
    dst2d = jnp.concatenate(
        [edge_dst.astype(jnp.int32), jnp.zeros((pad,), jnp.int32)]
    ).reshape(E_PAD // 128, 128)
    ecs4 = jnp.concatenate(
        [jnp.concatenate([edge_cell_shift,
                          jnp.zeros((E, 1), jnp.float32)], axis=1),
         jnp.zeros((pad, 4), jnp.float32)], axis=0)

    a_tab, x = _build_tables(atom_type, image_index, atom_pos, cell9, W_in)

    srcg, dstg = _sc_gather_geom(a_tab, src2d, dst2d)
    w_all = _edge_weights(srcg, dstg, ecs4, W_r1, W_r2, W_sh)

    for l in range(LAYERS):
        agg_pad = _sc_layer(x, w_all, l, src2d, dst2d)
        x = _node_update(x, agg_pad, W_sc[l], W_node[l])

    fermi, vac = _readout(image_index, x, W_v1, W_v2, W_f1, W_f2)
    return fermi, vac


# speedup vs baseline: 2.4945x; 1.1803x over previous
import functools

import jax
from jax import lax
import jax.numpy as jnp
from jax.experimental import pallas as pl
from jax.experimental.pallas import tpu as pltpu
from jax.experimental.pallas import tpu_sc as plsc

N = 50000
E = 800000
B = 64
ELEM = 2
D = 32
EMB = 16
NB = 8
LAYERS = 3
RMAX = 4.0
P = 6
AVG_NEI = 20.0
NEUR = 64
SH_DIM = 4

NBLK_N = 2000
NBLK_E = 4096

NC = 2
NS = 16
NW = NC * NS
CH = 512
RW = CH // 128
CPW = 50
E_PAD = NW * CH * CPW
ROWS_PW = CPW * RW
ZR = N // NS


def _silu(z):
    return z * jax.nn.sigmoid(z)


def _k1_body(at_ref, img_ref, pos_ref, cell9_ref, win_ref, a_ref, x0_ref):
    at = at_ref[0, 0, :]
    img = img_ref[0, 0, :]
    pos = pos_ref[...]
    blk = at.shape[0]
    oh = (jax.lax.broadcasted_iota(jnp.int32, (blk, B), 1) == img[:, None]).astype(jnp.float32)
    cellrow = jnp.dot(oh, cell9_ref[...], preferred_element_type=jnp.float32, precision=jax.lax.Precision.HIGHEST)
    a_ref[...] = jnp.concatenate(
        [pos, cellrow, jnp.zeros((blk, 4), jnp.float32)], axis=1)
    oh2 = (jax.lax.broadcasted_iota(jnp.int32, (blk, ELEM), 1) == at[:, None]).astype(jnp.float32)
    x0_ref[...] = jnp.dot(oh2, win_ref[...], preferred_element_type=jnp.float32)


def _build_tables(atom_type, image_index, atom_pos, cell9, W_in):
    nblk = N // NBLK_N
    at3 = atom_type.reshape(nblk, 1, NBLK_N).astype(jnp.int32)
    img3 = image_index.reshape(nblk, 1, NBLK_N).astype(jnp.int32)
    return pl.pallas_call(
        _k1_body,
        grid=(nblk,),
        in_specs=[
            pl.BlockSpec((1, 1, NBLK_N), lambda i: (i, 0, 0)),
            pl.BlockSpec((1, 1, NBLK_N), lambda i: (i, 0, 0)),
            pl.BlockSpec((NBLK_N, 3), lambda i: (i, 0)),
            pl.BlockSpec((B, 9), lambda i: (0, 0)),
            pl.BlockSpec((ELEM, D), lambda i: (0, 0)),
        ],
        out_specs=[
            pl.BlockSpec((NBLK_N, 16), lambda i: (i, 0)),
            pl.BlockSpec((NBLK_N, D), lambda i: (i, 0)),
        ],
        out_shape=[
            jax.ShapeDtypeStruct((N, 16), jnp.float32),
            jax.ShapeDtypeStruct((N, D), jnp.float32),
        ],
    )(at3, img3, atom_pos, cell9, W_in)


def _k3_body(srcg_ref, dstg_ref, ecs_ref, wr1_ref, wr2_ref, wsh_ref, w_ref):
    srcg = srcg_ref[...]
    dstg = dstg_ref[...]
    ecs = ecs_ref[...]
    be = srcg.shape[0]
    vs = []
    for j in range(3):
        shift = (ecs[:, 0:1] * srcg[:, 3 + j:4 + j]
                 + ecs[:, 1:2] * srcg[:, 6 + j:7 + j]
                 + ecs[:, 2:3] * srcg[:, 9 + j:10 + j])
        vs.append(dstg[:, j:j + 1] - srcg[:, j:j + 1] + shift)
    r2 = vs[0] * vs[0] + vs[1] * vs[1] + vs[2] * vs[2]
    r = jnp.sqrt(r2)
    rinv = 1.0 / (r + 1e-12)
    sh = jnp.concatenate(
        [jnp.ones((be, 1), jnp.float32),
         jnp.sqrt(3.0) * vs[0] * rinv,
         jnp.sqrt(3.0) * vs[1] * rinv,
         jnp.sqrt(3.0) * vs[2] * rinv], axis=1)
    u_full = r * (1.0 / RMAX)
    a = jnp.float32(jnp.pi) * jnp.minimum(u_full, 1.0)
    bb = jnp.minimum(a, jnp.float32(jnp.pi) - a)
    b2 = bb * bb
    sb = bb * (1.0 + b2 * (-1.0 / 6.0 + b2 * (1.0 / 120.0 + b2 * (-1.0 / 5040.0
         + b2 * (1.0 / 362880.0 - b2 * (1.0 / 39916800.0))))))
    cb = 1.0 + b2 * (-0.5 + b2 * (1.0 / 24.0 + b2 * (-1.0 / 720.0
         + b2 * (1.0 / 40320.0 + b2 * (-1.0 / 3628800.0 + b2 * (1.0 / 479001600.0))))))
    s1 = sb
    c1 = jnp.where(a > jnp.float32(jnp.pi / 2), -cb, cb)
    sn, cn = s1, c1
    sins = [s1]
    for _ in range(NB - 1):
        sn, cn = sn * c1 + cn * s1, cn * c1 - sn * s1
        sins.append(sn)
    sin_mat = jnp.concatenate(sins, axis=1)
    bes = jnp.sqrt(2.0 / RMAX) * sin_mat * rinv
    u = r / RMAX
    u2 = u * u
    u3 = u2 * u
    u6 = u3 * u3
    f_cut = (1.0 - ((P + 1.0) * (P + 2.0) / 2.0) * u6
             + P * (P + 2.0) * u6 * u - (P * (P + 1.0) / 2.0) * u6 * u2)
    f_cut = jnp.where(u < 1.0, f_cut, 0.0)
    radial = bes * f_cut
    row0 = pl.program_id(0) * be
    valid = ((jax.lax.broadcasted_iota(jnp.int32, (be, 1), 0) + row0) < E
             ).astype(jnp.float32)
    for l in range(LAYERS):
        h = _silu(jnp.dot(radial, wr1_ref[l], preferred_element_type=jnp.float32))
        rw = jnp.dot(h, wr2_ref[l], preferred_element_type=jnp.float32)
        eg = jnp.dot(sh, wsh_ref[l], preferred_element_type=jnp.float32)
        wl = rw * eg * ((1.0 / AVG_NEI) * valid)
        q = be // 4
        w_ref[l, :, :] = jnp.concatenate(
            [wl[0:q], wl[q:2 * q], wl[2 * q:3 * q], wl[3 * q:4 * q]], axis=1)


def _edge_weights(srcg, dstg, ecs4, W_r1, W_r2, W_sh):
    nblk = E_PAD // NBLK_E
    return pl.pallas_call(
        _k3_body,
        grid=(nblk,),
        in_specs=[
            pl.BlockSpec((NBLK_E, 16), lambda i: (i, 0)),
            pl.BlockSpec((NBLK_E, 16), lambda i: (i, 0)),
            pl.BlockSpec((NBLK_E, 4), lambda i: (i, 0)),
            pl.BlockSpec((LAYERS, NB, NEUR), lambda i: (0, 0, 0)),
            pl.BlockSpec((LAYERS, NEUR, D), lambda i: (0, 0, 0)),
            pl.BlockSpec((LAYERS, SH_DIM, D), lambda i: (0, 0, 0)),
        ],
        out_specs=pl.BlockSpec((LAYERS, NBLK_E // 4, 4 * D), lambda i: (0, i, 0)),
        out_shape=jax.ShapeDtypeStruct((LAYERS, E_PAD // 4, 4 * D), jnp.float32),
    )(srcg, dstg, ecs4, W_r1, W_r2, W_sh)


def _sc_gather_geom(a_tab, src2d, dst2d):
    mesh = plsc.VectorSubcoreMesh(core_axis_name="c", subcore_axis_name="s")

    @functools.partial(
        pl.kernel,
        out_type=[jax.ShapeDtypeStruct((E_PAD, 16), jnp.float32),
                  jax.ShapeDtypeStruct((E_PAD, 16), jnp.float32)],
        mesh=mesh,
        compiler_params=pltpu.CompilerParams(use_tc_tiling_on_sc=False),
        scratch_types=[
            pltpu.VMEM((RW, 128), jnp.int32),
            pltpu.VMEM((RW, 128), jnp.int32),
            pltpu.VMEM((CH, 16), jnp.float32),
            pltpu.VMEM((CH, 16), jnp.float32),
            pltpu.SemaphoreType.DMA,
            pltpu.SemaphoreType.DMA,
        ])
    def k(a_hbm, src_hbm, dst_hbm, srcg_hbm, dstg_hbm,
          idxs, idxd, bs, bd, sem_s, sem_d):
        wid = lax.axis_index("s") * NC + lax.axis_index("c")

        def body(c, _):
            row0 = wid * ROWS_PW + c * RW
            e0 = row0 * 128
            pltpu.sync_copy(src_hbm.at[pl.ds(row0, RW)], idxs)
            pltpu.sync_copy(dst_hbm.at[pl.ds(row0, RW)], idxd)
            cps = [pltpu.async_copy(a_hbm.at[idxs.at[j]],
                                    bs.at[pl.ds(j * 128, 128)], sem_s)
                   for j in range(RW)]
            cpd = [pltpu.async_copy(a_hbm.at[idxd.at[j]],
                                    bd.at[pl.ds(j * 128, 128)], sem_d)
                   for j in range(RW)]
            for cp in cps + cpd:
                cp.wait()
            pltpu.sync_copy(bs, srcg_hbm.at[pl.ds(e0, CH)])
            pltpu.sync_copy(bd, dstg_hbm.at[pl.ds(e0, CH)])
            return 0

        lax.fori_loop(0, CPW, body, 0)

    return k(a_tab, src2d, dst2d)


CHL = 256
CRW = CHL // 128
CPWL = E_PAD // (NW * CHL)


def _sc_layer(x, w_all, l, src2d, dst2d):
    mesh = plsc.VectorSubcoreMesh(core_axis_name="c", subcore_axis_name="s")

    @functools.partial(
        pl.kernel,
        out_type=jax.ShapeDtypeStruct((2 * N, D), jnp.float32),
        mesh=mesh,
        compiler_params=pltpu.CompilerParams(use_tc_tiling_on_sc=False),
        scratch_types=[
            pltpu.VMEM((CRW, 128), jnp.int32),
            pltpu.VMEM((CRW, 128), jnp.int32),
            pltpu.VMEM((CHL, D), jnp.float32),
            pltpu.VMEM((CHL, D), jnp.float32),
            pltpu.VMEM((128, D), jnp.float32),
            pltpu.VMEM_SHARED((N, D), jnp.float32),
            pltpu.SemaphoreType.DMA,
        ])
    def k(x_hbm, w_hbm, src_hbm, dst_hbm, out_hbm,
          idxs, idxd, xr, wr, zb, acc, sem):
        cid = lax.axis_index("c")
        sid = lax.axis_index("s")
        wid = sid * NC + cid

        def zvec(i, _):
            zb[i, pl.ds(0, 16)] = jnp.zeros((16,), jnp.float32)
            zb[i, pl.ds(16, 16)] = jnp.zeros((16,), jnp.float32)
            return 0
        lax.fori_loop(0, 128, zvec, 0, unroll=8)
        base = sid * ZR
        nfull = ZR // 128
        rem = ZR - nfull * 128
        def zcopy(i, _):
            pltpu.sync_copy(zb, acc.at[pl.ds(base + i * 128, 128)])
            return 0
        lax.fori_loop(0, nfull, zcopy, 0)
        if rem:
            pltpu.sync_copy(zb.at[pl.ds(0, rem)],
                            acc.at[pl.ds(base + nfull * 128, rem)])
        plsc.subcore_barrier()

        def body(c, _):
            row0 = wid * ROWS_PW + c * CRW
            e0 = row0 * 128
            pltpu.sync_copy(src_hbm.at[pl.ds(row0, CRW)], idxs)
            pltpu.sync_copy(dst_hbm.at[pl.ds(row0, CRW)], idxd)
            cps = [pltpu.async_copy(x_hbm.at[idxs.at[j]],
                                    xr.at[pl.ds(j * 128, 128)], sem)
                   for j in range(CRW)]
            blk = e0 // NBLK_E
            rbase = (e0 % NBLK_E) % (NBLK_E // 4)
            qb = (e0 % NBLK_E) // (NBLK_E // 4)
            pltpu.sync_copy(
                w_hbm.at[l,
                         pl.ds(blk * (NBLK_E // 4) + rbase, CHL),
                         pl.ds(qb * D, D)], wr)
            for cp in cps:
                cp.wait()
            def mul(i, _):
                xr[i, pl.ds(0, 16)] = xr[i, pl.ds(0, 16)] * wr[i, pl.ds(0, 16)]
                xr[i, pl.ds(16, 16)] = xr[i, pl.ds(16, 16)] * wr[i, pl.ds(16, 16)]
                return 0
            lax.fori_loop(0, CHL, mul, 0, unroll=8)
            for j in range(CRW):
                pltpu.sync_copy(xr.at[pl.ds(j * 128, 128)],
                                acc.at[idxd.at[j]], add=True)
            return 0

        lax.fori_loop(0, CPWL, body, 0)
        plsc.subcore_barrier()
        pltpu.sync_copy(acc.at[pl.ds(base, ZR)],
                        out_hbm.at[pl.ds(cid * N + base, ZR)])

    return k(x, w_all, src2d, dst2d)


def _k5_body(x_ref, p0_ref, p1_ref, wsc_ref, wnode_ref, out_ref):
    x = x_ref[...]
    agg = p0_ref[...] + p1_ref[...]
    sc = jnp.dot(x, wsc_ref[...], preferred_element_type=jnp.float32)
    new = _silu(jnp.dot(agg + sc, wnode_ref[...], preferred_element_type=jnp.float32))
    out_ref[...] = x + new


def _node_update(x, partial, wsc, wnode):
    nblk = N // NBLK_N
    return pl.pallas_call(
        _k5_body,
        grid=(nblk,),
        in_specs=[
            pl.BlockSpec((NBLK_N, D), lambda i: (i, 0)),
            pl.BlockSpec((NBLK_N, D), lambda i: (i, 0)),
            pl.BlockSpec((NBLK_N, D), lambda i: (i + nblk, 0)),
            pl.BlockSpec((D, D), lambda i: (0, 0)),
            pl.BlockSpec((D, D), lambda i: (0, 0)),
        ],
        out_specs=pl.BlockSpec((NBLK_N, D), lambda i: (i, 0)),
        out_shape=jax.ShapeDtypeStruct((N, D), jnp.float32),
    )(x, partial, partial, wsc, wnode)


def _k6_body(img_ref, x_ref, wv1_ref, wv2_ref, wf1_ref, wf2_ref,
             fermi_ref, vac_ref, s_ref, s2_ref, c_ref):
    i = pl.program_id(0)
    nsteps = pl.num_programs(0)

    @pl.when(i == 0)
    def _():
        s_ref[...] = jnp.zeros_like(s_ref)
        s2_ref[...] = jnp.zeros_like(s2_ref)
        c_ref[...] = jnp.zeros_like(c_ref)

    img = img_ref[0, 0, :]
    x = x_ref[...]
    blk = x.shape[0]
    oh = (jax.lax.broadcasted_iota(jnp.int32, (B, blk), 0) == img[None, :]).astype(jnp.float32)
    s_ref[...] += jnp.dot(oh, x, preferred_element_type=jnp.float32, precision=jax.lax.Precision.HIGHEST)
    s2_ref[...] += jnp.dot(oh, x * x, preferred_element_type=jnp.float32, precision=jax.lax.Precision.HIGHEST)
    c_ref[...] += jnp.sum(oh, axis=1, keepdims=True)

    @pl.when(i == nsteps - 1)
    def _():
        cnt = jnp.maximum(c_ref[...], 1.0)
        mean = s_ref[...] / cnt
        mean2 = s2_ref[...] / cnt
        std = jnp.sqrt(jnp.maximum(mean2 - mean * mean, 0.0) + 1e-12)
        fermi_ref[...] = jnp.dot(
            jnp.dot(std, wf1_ref[...], preferred_element_type=jnp.float32),
            wf2_ref[...], preferred_element_type=jnp.float32)
        vac_ref[...] = jnp.dot(
            jnp.dot(mean, wv1_ref[...], preferred_element_type=jnp.float32),
            wv2_ref[...], preferred_element_type=jnp.float32)


def _readout(image_index, x, W_v1, W_v2, W_f1, W_f2):
    nblk = N // NBLK_N
    img3 = image_index.reshape(nblk, 1, NBLK_N).astype(jnp.int32)
    return pl.pallas_call(
        _k6_body,
        grid=(nblk,),
        in_specs=[
            pl.BlockSpec((1, 1, NBLK_N), lambda i: (i, 0, 0)),
            pl.BlockSpec((NBLK_N, D), lambda i: (i, 0)),
            pl.BlockSpec((D, EMB), lambda i: (0, 0)),
            pl.BlockSpec((EMB, 1), lambda i: (0, 0)),
            pl.BlockSpec((D, EMB), lambda i: (0, 0)),
            pl.BlockSpec((EMB, 1), lambda i: (0, 0)),
        ],
        out_specs=[
            pl.BlockSpec((B, 1), lambda i: (0, 0)),
            pl.BlockSpec((B, 1), lambda i: (0, 0)),
        ],
        out_shape=[
            jax.ShapeDtypeStruct((B, 1), jnp.float32),
            jax.ShapeDtypeStruct((B, 1), jnp.float32),
        ],
        scratch_shapes=[
            pltpu.VMEM((B, D), jnp.float32),
            pltpu.VMEM((B, D), jnp.float32),
            pltpu.VMEM((B, 1), jnp.float32),
        ],
    )(img3, x, W_v1, W_v2, W_f1, W_f2)


def kernel(atom_type, atom_pos, edge_src, edge_dst, edge_cell_shift, cell,
           image_index, W_in, W_r1, W_r2, W_sh, W_sc, W_node, W_v1, W_v2,
           W_f1, W_f2):
    atom_type = atom_type.astype(jnp.int32)
    image_index = image_index.astype(jnp.int32)
    cell9 = cell.reshape(B, 9).astype(jnp.float32)
    pad = E_PAD - E
    src2d = jnp.concatenate(
        [edge_src.astype(jnp.int32), jnp.zeros((pad,), jnp.int32)]
    ).reshape(E_PAD // 128, 128)
    dst2d = jnp.concatenate(
        [edge_dst.astype(jnp.int32), jnp.zeros((pad,), jnp.int32)]
    ).reshape(E_PAD // 128, 128)
    ecs4 = jnp.concatenate(
        [jnp.concatenate([edge_cell_shift,
                          jnp.zeros((E, 1), jnp.float32)], axis=1),
         jnp.zeros((pad, 4), jnp.float32)], axis=0)

    a_tab, x = _build_tables(atom_type, image_index, atom_pos, cell9, W_in)

    srcg, dstg = _sc_gather_geom(a_tab, src2d, dst2d)
    w_all = _edge_weights(srcg, dstg, ecs4, W_r1, W_r2, W_sh)

    for l in range(LAYERS):
        partial = _sc_layer(x, w_all, l, src2d, dst2d)
        x = _node_update(x, partial, W_sc[l], W_node[l])

    fermi, vac = _readout(image_index, x, W_v1, W_v2, W_f1, W_f2)
    return fermi, vac
